# baseline (speedup 1.0000x reference)
import functools

import jax
import jax.numpy as jnp
from jax import lax
from jax.experimental import pallas as pl
from jax.experimental.pallas import tpu as pltpu
from jax.experimental.pallas import tpu_sc as plsc

N_NEURONS = 100000
DT = 0.1
DECAY = 1.0 - DT

NC = 2
NS = 16
NW = NC * NS
L = 16

TPT = 3136
NPAD = NW * TPT
TBITS = 13
TMASK = (1 << TBITS) - 1
CHUNK = 2048
UNROLL = 16
SUB = CHUNK // L // UNROLL


def _step(st_hbm, w_hbm, starts_hbm, m_hbm, tm1_hbm, v_hbm, r_hbm,
          v_out, r_out,
          r_full, acc, vsl, msl, tsl, rsl, stb0, wb0, stb1, wb1, stv,
          sem0, sem1):
    wid = lax.axis_index("c") * NS + lax.axis_index("s")
    off0 = pl.multiple_of(wid * TPT, 8)

    rcp = pltpu.make_async_copy(r_hbm, r_full, sem1)
    rcp.start()
    pltpu.sync_copy(starts_hbm, stv)
    pltpu.sync_copy(v_hbm.at[pl.ds(off0, TPT)], vsl)
    pltpu.sync_copy(m_hbm.at[pl.ds(off0, TPT)], msl)
    pltpu.sync_copy(tm1_hbm.at[pl.ds(off0, TPT)], tsl)

    start = jnp.max(plsc.load_gather(stv, [jnp.full((L,), wid, jnp.int32)]))
    end = jnp.max(plsc.load_gather(stv, [jnp.full((L,), wid + 1, jnp.int32)]))
    base = jnp.bitwise_and(start, jnp.int32(-8))
    nch = (end - base + (CHUNK - 1)) // CHUNK

    zeros = jnp.zeros((L,), jnp.float32)

    def _zero(i, carry):
        acc[pl.ds(i * L, L)] = zeros
        return carry

    lax.fori_loop(0, TPT // L, _zero, 0)

    iota = lax.broadcasted_iota(jnp.int32, (L,), 0)
    rcp.wait()

    def _process(stbuf, wbuf, off):
        def _inner(k, c):
            o = k * (UNROLL * L)
            for u in range(UNROLL):
                oo = o + u * L
                st = stbuf[pl.ds(oo, L)]
                sv = jnp.minimum(
                    lax.shift_right_logical(st, TBITS), jnp.int32(NPAD - 1))
                tv = jnp.minimum(
                    jnp.bitwise_and(st, jnp.int32(TMASK)), jnp.int32(TPT - 1))
                wv = wbuf[pl.ds(oo, L)]
                pos = iota + (off + oo)
                ok = jnp.logical_and(pos >= start, pos < end)
                wm = jnp.where(ok, wv, 0.0)
                vals = plsc.load_gather(r_full, [sv])
                plsc.addupdate_scatter(acc, [tv], vals * wm)
            return c

        lax.fori_loop(0, SUB, _inner, 0)

    offp = pl.multiple_of(base, 8)
    pltpu.make_async_copy(st_hbm.at[pl.ds(offp, CHUNK)], stb0, sem0).start()
    pltpu.make_async_copy(w_hbm.at[pl.ds(offp, CHUNK)], wb0, sem0).start()

    def _chunk2(p, c):
        offa = pl.multiple_of(base + (2 * p) * CHUNK, 8)
        offb = pl.multiple_of(base + (2 * p + 1) * CHUNK, 8)
        offc = pl.multiple_of(base + (2 * p + 2) * CHUNK, 8)
        pltpu.make_async_copy(st_hbm.at[pl.ds(offa, CHUNK)], stb0, sem0).wait()
        pltpu.make_async_copy(w_hbm.at[pl.ds(offa, CHUNK)], wb0, sem0).wait()
        pltpu.make_async_copy(st_hbm.at[pl.ds(offb, CHUNK)], stb1, sem1).start()
        pltpu.make_async_copy(w_hbm.at[pl.ds(offb, CHUNK)], wb1, sem1).start()
        _process(stb0, wb0, offa)
        pltpu.make_async_copy(st_hbm.at[pl.ds(offb, CHUNK)], stb1, sem1).wait()
        pltpu.make_async_copy(w_hbm.at[pl.ds(offb, CHUNK)], wb1, sem1).wait()
        pltpu.make_async_copy(st_hbm.at[pl.ds(offc, CHUNK)], stb0, sem0).start()
        pltpu.make_async_copy(w_hbm.at[pl.ds(offc, CHUNK)], wb0, sem0).start()
        _process(stb1, wb1, offb)
        return c

    npairs = (nch + 1) // 2
    lax.fori_loop(0, npairs, _chunk2, 0)
    offz = pl.multiple_of(base + 2 * npairs * CHUNK, 8)
    pltpu.make_async_copy(st_hbm.at[pl.ds(offz, CHUNK)], stb0, sem0).wait()
    pltpu.make_async_copy(w_hbm.at[pl.ds(offz, CHUNK)], wb0, sem0).wait()

    def _upd(i, carry):
        ds = pl.ds(i * L, L)
        syn = acc[ds]
        v = vsl[ds]
        m = msl[ds]
        t = tsl[ds]
        vn = v * DECAY + syn * DT
        vn = vn * (1.0 - m) + t * m
        vsl[ds] = vn
        rsl[ds] = jnp.maximum(vn, 0.0)
        return carry

    lax.fori_loop(0, TPT // L, _upd, 0)

    pltpu.sync_copy(vsl, v_out.at[pl.ds(off0, TPT)])
    pltpu.sync_copy(rsl, r_out.at[pl.ds(off0, TPT)])


_step_call = functools.partial(
    pl.kernel,
    out_type=(
        jax.ShapeDtypeStruct((NPAD,), jnp.float32),
        jax.ShapeDtypeStruct((NPAD,), jnp.float32),
    ),
    mesh=plsc.VectorSubcoreMesh(
        core_axis_name="c", subcore_axis_name="s", num_cores=NC,
        num_subcores=NS,
    ),
    compiler_params=pltpu.CompilerParams(needs_layout_passes=False),
    scratch_types=(
        pltpu.VMEM((NPAD,), jnp.float32),
        pltpu.VMEM((TPT,), jnp.float32),
        pltpu.VMEM((TPT,), jnp.float32),
        pltpu.VMEM((TPT,), jnp.float32),
        pltpu.VMEM((TPT,), jnp.float32),
        pltpu.VMEM((TPT,), jnp.float32),
        pltpu.VMEM((CHUNK,), jnp.int32),
        pltpu.VMEM((CHUNK,), jnp.float32),
        pltpu.VMEM((CHUNK,), jnp.int32),
        pltpu.VMEM((CHUNK,), jnp.float32),
        pltpu.VMEM((48,), jnp.int32),
        pltpu.SemaphoreType.DMA,
        pltpu.SemaphoreType.DMA,
    ),
)(_step)


PC = 2048


def _make_permute(pp, e_pad):
    pt = pp // NW
    nch = pt // PC

    def body(st_hbm, w_hbm, dst_hbm, sto, wo,
             sb0, wb0, db0, sb1, wb1, db1, semi0, semi1, semo0, semo1):
        wid = lax.axis_index("c") * NS + lax.axis_index("s")
        base = pl.multiple_of(wid * pt, 8)
        bufs = ((sb0, wb0, db0, semi0, semo0), (sb1, wb1, db1, semi1, semo1))

        def start_in(j, bb):
            sb, wb, db, semi, _ = bb
            off = pl.multiple_of(base + j * PC, 8)
            pltpu.make_async_copy(st_hbm.at[pl.ds(off, PC)], sb, semi).start()
            pltpu.make_async_copy(w_hbm.at[pl.ds(off, PC)], wb, semi).start()
            pltpu.make_async_copy(dst_hbm.at[pl.ds(off, PC)], db, semi).start()

        def wait_in(j, bb):
            sb, wb, db, semi, _ = bb
            off = pl.multiple_of(base + j * PC, 8)
            pltpu.make_async_copy(st_hbm.at[pl.ds(off, PC)], sb, semi).wait()
            pltpu.make_async_copy(w_hbm.at[pl.ds(off, PC)], wb, semi).wait()
            pltpu.make_async_copy(dst_hbm.at[pl.ds(off, PC)], db, semi).wait()

        def start_scat(bb):
            sb, wb, db, _, semo = bb
            pltpu.make_async_copy(sb, sto.at[db], semo).start()
            pltpu.make_async_copy(wb, wo.at[db], semo).start()

        def wait_scat(bb):
            sb, wb, db, _, semo = bb
            pltpu.make_async_copy(sb, sto.at[db], semo).wait()
            pltpu.make_async_copy(wb, wo.at[db], semo).wait()

        start_in(0, bufs[0])
        for j in range(nch):
            cur, nxt = bufs[j % 2], bufs[(j + 1) % 2]
            wait_in(j, cur)
            start_scat(cur)
            if j + 1 < nch:
                if j >= 1:
                    wait_scat(nxt)
                start_in(j + 1, nxt)
        if nch >= 2:
            wait_scat(bufs[(nch - 2) % 2])
        wait_scat(bufs[(nch - 1) % 2])

    return pl.kernel(
        body,
        out_type=(
            jax.ShapeDtypeStruct((e_pad,), jnp.int32),
            jax.ShapeDtypeStruct((e_pad,), jnp.float32),
        ),
        mesh=plsc.VectorSubcoreMesh(
            core_axis_name="c", subcore_axis_name="s", num_cores=NC,
            num_subcores=NS,
        ),
        compiler_params=pltpu.CompilerParams(needs_layout_passes=False),
        scratch_types=(
            pltpu.VMEM((PC,), jnp.int32),
            pltpu.VMEM((PC,), jnp.float32),
            pltpu.VMEM((PC,), jnp.int32),
            pltpu.VMEM((PC,), jnp.int32),
            pltpu.VMEM((PC,), jnp.float32),
            pltpu.VMEM((PC,), jnp.int32),
            pltpu.SemaphoreType.DMA,
            pltpu.SemaphoreType.DMA,
            pltpu.SemaphoreType.DMA,
            pltpu.SemaphoreType.DMA,
        ),
    )


def _partition_by_target(st, w, bucket, e_pad):
    e = bucket.shape[0]
    S = 128
    k = -(-e // S)
    ep = k * S
    nb = NW + 1
    bp = jnp.concatenate(
        [bucket, jnp.full((ep - e,), NW, jnp.int32)]).reshape(k, S)
    oh = (bp[:, :, None] == jnp.arange(nb, dtype=jnp.int32)).astype(
        jnp.bfloat16)
    slt = (jnp.arange(S)[:, None] > jnp.arange(S)[None, :]).astype(
        jnp.bfloat16)
    ranks = jnp.einsum(
        "st,ktb->ksb", slt, oh,
        preferred_element_type=jnp.float32)
    rank_pe = jnp.einsum(
        "ksb,ksb->ks", ranks, oh.astype(jnp.float32),
        preferred_element_type=jnp.float32,
        precision=lax.Precision.HIGHEST)
    seg_counts = jnp.sum(oh.astype(jnp.float32), axis=1)
    seg_off = jnp.cumsum(seg_counts, axis=0) - seg_counts
    totals = jnp.sum(seg_counts, axis=0)
    offsets = jnp.cumsum(totals) - totals
    base_pe = jnp.einsum(
        "kb,ksb->ks", seg_off + offsets[None, :], oh.astype(jnp.float32),
        preferred_element_type=jnp.float32,
        precision=lax.Precision.HIGHEST)
    dst = (base_pe + rank_pe).astype(jnp.int32).reshape(ep)[:e]
    pt = -(-ep // (NW * PC)) * PC
    pp = NW * pt
    st_p = jnp.zeros((pp,), jnp.int32).at[:e].set(st)
    w_p = jnp.zeros((pp,), jnp.float32).at[:e].set(w)
    dst_p = jnp.full((pp,), e, jnp.int32).at[:e].set(dst)
    st_e, w_e = _make_permute(pp, e_pad)(st_p, w_p, dst_p)
    starts = offsets.astype(jnp.int32)
    return st_e, w_e, starts


def kernel(tm1_input, weights, source_indices, target_indices, type_ids, steps):
    e = weights.shape[0]
    e_pad = e + 4 * CHUNK + 128

    bucket = target_indices // TPT
    st = source_indices * (1 << TBITS) + jnp.remainder(target_indices, TPT)
    st_e, w_e, starts = _partition_by_target(st, weights, bucket, e_pad)
    starts = jnp.concatenate(
        [starts, jnp.full((48 - NW - 1,), e, dtype=jnp.int32)])

    tm1_idx = jnp.nonzero(type_ids == 0, size=tm1_input.shape[1])[0]
    m = jnp.zeros((NPAD,), jnp.float32).at[tm1_idx].set(1.0)
    tm1f = jnp.zeros((NPAD,), jnp.float32).at[tm1_idx].set(tm1_input[0])
    v0 = tm1f
    r0 = jnp.maximum(v0, 0.0)

    def body(_, carry):
        v, r = carry
        return _step_call(st_e, w_e, starts, m, tm1f, v, r)

    v_fin, _ = lax.fori_loop(0, steps, body, (v0, r0))
    return v_fin[:N_NEURONS].reshape(1, N_NEURONS)


# speedup vs baseline: 6.4014x; 6.4014x over previous
import functools

import jax
import jax.numpy as jnp
from jax import lax
from jax.experimental import pallas as pl
from jax.experimental.pallas import tpu as pltpu
from jax.experimental.pallas import tpu_sc as plsc

N_NEURONS = 100000
DT = 0.1
DECAY = 1.0 - DT

NC = 2
NS = 16
NW = NC * NS
L = 16

TPT = 3136
NPAD = NW * TPT
SPT = NPAD // NS
CHUNK = 2048
UNROLL = 16


def _make_scatter(e):
    assert e % NW == 0 and (e // NW) % L == 0
    pt = e // NW
    nch = -(-pt // CHUNK)

    def body(s_hbm, t_hbm, w_hbm, r_hbm, p_out,
             r_full, acc, zb, sb0, tb0, wb0, cb0, sb1, tb1, wb1, cb1,
             semr, semi0, semi1, semo0, semo1):
        cid = lax.axis_index("c")
        sid = lax.axis_index("s")
        wid = cid * NS + sid
        base = pl.multiple_of(wid * pt, 8)

        rcp = pltpu.make_async_copy(r_hbm, r_full, semr)
        rcp.start()

        zeros = jnp.zeros((L,), jnp.float32)

        def _z(i, c):
            zb[pl.ds(i * L, L)] = zeros
            return c

        lax.fori_loop(0, SPT // L, _z, 0)
        zoff = pl.multiple_of(sid * SPT, 8)
        pltpu.sync_copy(zb, acc.at[pl.ds(zoff, SPT)])
        plsc.subcore_barrier()
        rcp.wait()

        bufs = (
            (sb0, tb0, wb0, cb0, semi0, semo0),
            (sb1, tb1, wb1, cb1, semi1, semo1),
        )

        def start_in(j, bb):
            sb, tb, wb, _, semi, _ = bb
            off = pl.multiple_of(base + j * CHUNK, 8)
            n = min(CHUNK, pt - j * CHUNK)
            pltpu.make_async_copy(s_hbm.at[pl.ds(off, n)],
                                  sb.at[pl.ds(0, n)], semi).start()
            pltpu.make_async_copy(t_hbm.at[pl.ds(off, n)],
                                  tb.at[pl.ds(0, n)], semi).start()
            pltpu.make_async_copy(w_hbm.at[pl.ds(off, n)],
                                  wb.at[pl.ds(0, n)], semi).start()

        def wait_in(j, bb):
            sb, tb, wb, _, semi, _ = bb
            off = pl.multiple_of(base + j * CHUNK, 8)
            n = min(CHUNK, pt - j * CHUNK)
            pltpu.make_async_copy(s_hbm.at[pl.ds(off, n)],
                                  sb.at[pl.ds(0, n)], semi).wait()
            pltpu.make_async_copy(t_hbm.at[pl.ds(off, n)],
                                  tb.at[pl.ds(0, n)], semi).wait()
            pltpu.make_async_copy(w_hbm.at[pl.ds(off, n)],
                                  wb.at[pl.ds(0, n)], semi).wait()

        zeros16 = jnp.zeros((L,), jnp.float32)

        def compute(j, bb):
            sb, _, wb, cb, _, _ = bb
            n = min(CHUNK, pt - j * CHUNK)
            nv = n // L

            def _g(k, c):
                o = k * (UNROLL * L)
                for u in range(UNROLL):
                    oo = o + u * L
                    sv = sb[pl.ds(oo, L)]
                    wv = wb[pl.ds(oo, L)]
                    vals = plsc.load_gather(r_full, [sv])
                    cb[pl.ds(oo, L)] = vals * wv
                return c

            lax.fori_loop(0, nv // UNROLL, _g, 0)
            for u in range(nv - (nv // UNROLL) * UNROLL):
                oo = (nv // UNROLL) * UNROLL * L + u * L
                sv = sb[pl.ds(oo, L)]
                wv = wb[pl.ds(oo, L)]
                vals = plsc.load_gather(r_full, [sv])
                cb[pl.ds(oo, L)] = vals * wv
            if n < CHUNK:
                def _zt(k, c):
                    cb[pl.ds(n + k * L, L)] = zeros16
                    return c

                lax.fori_loop(0, (CHUNK - n) // L, _zt, 0)

        def start_scat(bb):
            _, tb, _, cb, _, semo = bb
            pltpu.async_copy(cb, acc.at[tb], semo, add=True)

        def wait_scat(bb):
            _, tb, _, cb, _, semo = bb
            pltpu.make_async_copy(cb, acc.at[tb], semo).wait()

        start_in(0, bufs[0])
        for j in range(nch):
            cur, nxt = bufs[j % 2], bufs[(j + 1) % 2]
            wait_in(j, cur)
            if j + 1 < nch:
                if j >= 1:
                    wait_scat(nxt)
                start_in(j + 1, nxt)
            compute(j, cur)
            start_scat(cur)
        if nch >= 2:
            wait_scat(bufs[(nch - 2) % 2])
        wait_scat(bufs[(nch - 1) % 2])

        plsc.subcore_barrier()
        poff = pl.multiple_of(cid * NPAD + sid * SPT, 8)
        pltpu.sync_copy(acc.at[pl.ds(zoff, SPT)], p_out.at[pl.ds(poff, SPT)])

    return pl.kernel(
        body,
        out_type=jax.ShapeDtypeStruct((NC * NPAD,), jnp.float32),
        mesh=plsc.VectorSubcoreMesh(
            core_axis_name="c", subcore_axis_name="s", num_cores=NC,
            num_subcores=NS,
        ),
        compiler_params=pltpu.CompilerParams(needs_layout_passes=False),
        scratch_types=(
            pltpu.VMEM((NPAD,), jnp.float32),
            pltpu.VMEM_SHARED((NPAD,), jnp.float32),
            pltpu.VMEM((SPT,), jnp.float32),
            pltpu.VMEM((CHUNK,), jnp.int32),
            pltpu.VMEM((CHUNK,), jnp.int32),
            pltpu.VMEM((CHUNK,), jnp.float32),
            pltpu.VMEM((CHUNK,), jnp.float32),
            pltpu.VMEM((CHUNK,), jnp.int32),
            pltpu.VMEM((CHUNK,), jnp.int32),
            pltpu.VMEM((CHUNK,), jnp.float32),
            pltpu.VMEM((CHUNK,), jnp.float32),
            pltpu.SemaphoreType.DMA,
            pltpu.SemaphoreType.DMA,
            pltpu.SemaphoreType.DMA,
            pltpu.SemaphoreType.DMA,
            pltpu.SemaphoreType.DMA,
        ),
    )


def _update(p_hbm, m_hbm, tm1_hbm, v_hbm, v_out, r_out,
            p0, p1, vsl, msl, tsl, rsl):
    wid = lax.axis_index("c") * NS + lax.axis_index("s")
    off0 = pl.multiple_of(wid * TPT, 8)
    pltpu.sync_copy(p_hbm.at[pl.ds(off0, TPT)], p0)
    pltpu.sync_copy(p_hbm.at[pl.ds(pl.multiple_of(NPAD + wid * TPT, 8), TPT)],
                    p1)
    pltpu.sync_copy(v_hbm.at[pl.ds(off0, TPT)], vsl)
    pltpu.sync_copy(m_hbm.at[pl.ds(off0, TPT)], msl)
    pltpu.sync_copy(tm1_hbm.at[pl.ds(off0, TPT)], tsl)

    def _u(i, c):
        ds = pl.ds(i * L, L)
        syn = p0[ds] + p1[ds]
        vn = vsl[ds] * DECAY + syn * DT
        m = msl[ds]
        vn = vn * (1.0 - m) + tsl[ds] * m
        vsl[ds] = vn
        rsl[ds] = jnp.maximum(vn, 0.0)
        return c

    lax.fori_loop(0, TPT // L, _u, 0)
    pltpu.sync_copy(vsl, v_out.at[pl.ds(off0, TPT)])
    pltpu.sync_copy(rsl, r_out.at[pl.ds(off0, TPT)])


_update_call = functools.partial(
    pl.kernel,
    out_type=(
        jax.ShapeDtypeStruct((NPAD,), jnp.float32),
        jax.ShapeDtypeStruct((NPAD,), jnp.float32),
    ),
    mesh=plsc.VectorSubcoreMesh(
        core_axis_name="c", subcore_axis_name="s", num_cores=NC,
        num_subcores=NS,
    ),
    compiler_params=pltpu.CompilerParams(needs_layout_passes=False),
    scratch_types=(
        pltpu.VMEM((TPT,), jnp.float32),
        pltpu.VMEM((TPT,), jnp.float32),
        pltpu.VMEM((TPT,), jnp.float32),
        pltpu.VMEM((TPT,), jnp.float32),
        pltpu.VMEM((TPT,), jnp.float32),
        pltpu.VMEM((TPT,), jnp.float32),
    ),
)(_update)


def kernel(tm1_input, weights, source_indices, target_indices, type_ids, steps):
    e = weights.shape[0]
    scatter_call = _make_scatter(e)

    tm1_idx = jnp.nonzero(type_ids == 0, size=tm1_input.shape[1])[0]
    m = jnp.zeros((NPAD,), jnp.float32).at[tm1_idx].set(1.0)
    tm1f = jnp.zeros((NPAD,), jnp.float32).at[tm1_idx].set(tm1_input[0])
    v0 = tm1f
    r0 = jnp.maximum(v0, 0.0)

    def body(_, carry):
        v, r = carry
        p = scatter_call(source_indices, target_indices, weights, r)
        return _update_call(p, m, tm1f, v)

    v_fin, _ = lax.fori_loop(0, steps, body, (v0, r0))
    return v_fin[:N_NEURONS].reshape(1, N_NEURONS)


# speedup vs baseline: 6.8015x; 1.0625x over previous
import functools

import jax
import jax.numpy as jnp
from jax import lax
from jax.experimental import pallas as pl
from jax.experimental.pallas import tpu as pltpu
from jax.experimental.pallas import tpu_sc as plsc

N_NEURONS = 100000
DT = 0.1
DECAY = 1.0 - DT

NC = 2
NS = 16
NW = NC * NS
L = 16

TPT = 3136
NPAD = NW * TPT
SPT = NPAD // NS
CHUNK = 2048
UNROLL = 16


def _make_step(e):
    assert e % NW == 0 and (e // NW) % L == 0
    pt = e // NW
    nch = -(-pt // CHUNK)
    assert nch >= 2 and pt - (nch - 1) * CHUNK >= L

    def body(s_hbm, t_hbm, w_hbm, p_hbm, m_hbm, tm1_hbm, v_hbm,
             v_out, p_out, rb_hbm,
             r_full, acc, sb0, tb0, wb0, cb0, sb1, tb1, wb1, cb1,
             semr, semi0, semi1, semo0, semo1):
        cid = lax.axis_index("c")
        sid = lax.axis_index("s")
        wid = cid * NS + sid
        base = pl.multiple_of(wid * pt, 8)
        uoff = pl.multiple_of(sid * SPT, 8)

        P0, P1, VV, MM, TT, RR, ZZ = (
            r_full.at[pl.ds(k * SPT, SPT)] for k in range(7))

        pltpu.make_async_copy(p_hbm.at[pl.ds(uoff, SPT)], P0, semr).start()
        pltpu.make_async_copy(
            p_hbm.at[pl.ds(pl.multiple_of(NPAD + sid * SPT, 8), SPT)],
            P1, semr).start()
        pltpu.make_async_copy(v_hbm.at[pl.ds(uoff, SPT)], VV, semr).start()
        pltpu.make_async_copy(m_hbm.at[pl.ds(uoff, SPT)], MM, semr).start()
        pltpu.make_async_copy(tm1_hbm.at[pl.ds(uoff, SPT)], TT, semr).start()

        zeros = jnp.zeros((L,), jnp.float32)

        def _z(i, c):
            ZZ[pl.ds(i * L, L)] = zeros
            return c

        lax.fori_loop(0, SPT // L, _z, 0)
        pltpu.sync_copy(ZZ, acc.at[pl.ds(uoff, SPT)])

        pltpu.make_async_copy(p_hbm.at[pl.ds(uoff, SPT)], P0, semr).wait()
        pltpu.make_async_copy(
            p_hbm.at[pl.ds(pl.multiple_of(NPAD + sid * SPT, 8), SPT)],
            P1, semr).wait()
        pltpu.make_async_copy(v_hbm.at[pl.ds(uoff, SPT)], VV, semr).wait()
        pltpu.make_async_copy(m_hbm.at[pl.ds(uoff, SPT)], MM, semr).wait()
        pltpu.make_async_copy(tm1_hbm.at[pl.ds(uoff, SPT)], TT, semr).wait()

        def _u(i, c):
            ds = pl.ds(i * L, L)
            syn = P0[ds] + P1[ds]
            vn = VV[ds] * DECAY + syn * DT
            m = MM[ds]
            vn = vn * (1.0 - m) + TT[ds] * m
            VV[ds] = vn
            RR[ds] = jnp.maximum(vn, 0.0)
            return c

        lax.fori_loop(0, SPT // L, _u, 0)

        @pl.when(cid == 0)
        def _():
            pltpu.sync_copy(VV, v_out.at[pl.ds(uoff, SPT)])

        roff = pl.multiple_of(cid * NPAD + sid * SPT, 8)
        pltpu.sync_copy(RR, rb_hbm.at[pl.ds(roff, SPT)])
        plsc.subcore_barrier()

        rcp = pltpu.make_async_copy(
            rb_hbm.at[pl.ds(pl.multiple_of(cid * NPAD, 8), NPAD)],
            r_full, semr)
        rcp.start()

        bufs = (
            (sb0, tb0, wb0, cb0, semi0, semo0),
            (sb1, tb1, wb1, cb1, semi1, semo1),
        )

        def start_in(j, bb):
            sb, tb, wb, _, semi, _ = bb
            off = pl.multiple_of(base + j * CHUNK, 8)
            n = min(CHUNK, pt - j * CHUNK)
            pltpu.make_async_copy(s_hbm.at[pl.ds(off, n)],
                                  sb.at[pl.ds(0, n)], semi).start()
            pltpu.make_async_copy(t_hbm.at[pl.ds(off, n)],
                                  tb.at[pl.ds(0, n)], semi).start()
            pltpu.make_async_copy(w_hbm.at[pl.ds(off, n)],
                                  wb.at[pl.ds(0, n)], semi).start()

        def wait_in(j, bb):
            sb, tb, wb, _, semi, _ = bb
            off = pl.multiple_of(base + j * CHUNK, 8)
            n = min(CHUNK, pt - j * CHUNK)
            pltpu.make_async_copy(s_hbm.at[pl.ds(off, n)],
                                  sb.at[pl.ds(0, n)], semi).wait()
            pltpu.make_async_copy(t_hbm.at[pl.ds(off, n)],
                                  tb.at[pl.ds(0, n)], semi).wait()
            pltpu.make_async_copy(w_hbm.at[pl.ds(off, n)],
                                  wb.at[pl.ds(0, n)], semi).wait()

        def compute(j, bb):
            sb, _, wb, cb, _, _ = bb
            n = min(CHUNK, pt - j * CHUNK)
            nv = n // L

            def _g(k, c):
                o = k * (UNROLL * L)
                for u in range(UNROLL):
                    oo = o + u * L
                    sv = sb[pl.ds(oo, L)]
                    wv = wb[pl.ds(oo, L)]
                    vals = plsc.load_gather(r_full, [sv])
                    cb[pl.ds(oo, L)] = vals * wv
                return c

            lax.fori_loop(0, nv // UNROLL, _g, 0)
            for u in range(nv - (nv // UNROLL) * UNROLL):
                oo = (nv // UNROLL) * UNROLL * L + u * L
                sv = sb[pl.ds(oo, L)]
                wv = wb[pl.ds(oo, L)]
                vals = plsc.load_gather(r_full, [sv])
                cb[pl.ds(oo, L)] = vals * wv
            if n < CHUNK:
                def _zt(k, c):
                    cb[pl.ds(n + k * L, L)] = zeros
                    return c

                lax.fori_loop(0, (CHUNK - n) // L, _zt, 0)

        def start_scat(bb):
            _, tb, _, cb, _, semo = bb
            pltpu.async_copy(cb, acc.at[tb], semo, add=True)

        def wait_scat(bb):
            _, tb, _, cb, _, semo = bb
            pltpu.make_async_copy(cb, acc.at[tb], semo).wait()

        start_in(0, bufs[0])
        rcp.wait()
        for j in range(nch):
            cur, nxt = bufs[j % 2], bufs[(j + 1) % 2]
            wait_in(j, cur)
            if j + 1 < nch:
                if j >= 1:
                    wait_scat(nxt)
                start_in(j + 1, nxt)
            compute(j, cur)
            start_scat(cur)
        wait_scat(bufs[(nch - 2) % 2])
        wait_scat(bufs[(nch - 1) % 2])

        plsc.subcore_barrier()
        poff = pl.multiple_of(cid * NPAD + sid * SPT, 8)
        pltpu.sync_copy(acc.at[pl.ds(uoff, SPT)], p_out.at[pl.ds(poff, SPT)])

    return pl.kernel(
        body,
        out_type=(
            jax.ShapeDtypeStruct((NPAD,), jnp.float32),
            jax.ShapeDtypeStruct((NC * NPAD,), jnp.float32),
            jax.ShapeDtypeStruct((NC * NPAD,), jnp.float32),
        ),
        mesh=plsc.VectorSubcoreMesh(
            core_axis_name="c", subcore_axis_name="s", num_cores=NC,
            num_subcores=NS,
        ),
        compiler_params=pltpu.CompilerParams(needs_layout_passes=False),
        scratch_types=(
            pltpu.VMEM((NPAD,), jnp.float32),
            pltpu.VMEM_SHARED((NPAD,), jnp.float32),
            pltpu.VMEM((CHUNK,), jnp.int32),
            pltpu.VMEM((CHUNK,), jnp.int32),
            pltpu.VMEM((CHUNK,), jnp.float32),
            pltpu.VMEM((CHUNK,), jnp.float32),
            pltpu.VMEM((CHUNK,), jnp.int32),
            pltpu.VMEM((CHUNK,), jnp.int32),
            pltpu.VMEM((CHUNK,), jnp.float32),
            pltpu.VMEM((CHUNK,), jnp.float32),
            pltpu.SemaphoreType.DMA,
            pltpu.SemaphoreType.DMA,
            pltpu.SemaphoreType.DMA,
            pltpu.SemaphoreType.DMA,
            pltpu.SemaphoreType.DMA,
        ),
    )


def _update(p_hbm, m_hbm, tm1_hbm, v_hbm, v_out,
            p0, p1, vsl, msl, tsl):
    wid = lax.axis_index("c") * NS + lax.axis_index("s")
    off0 = pl.multiple_of(wid * TPT, 8)
    pltpu.sync_copy(p_hbm.at[pl.ds(off0, TPT)], p0)
    pltpu.sync_copy(p_hbm.at[pl.ds(pl.multiple_of(NPAD + wid * TPT, 8), TPT)],
                    p1)
    pltpu.sync_copy(v_hbm.at[pl.ds(off0, TPT)], vsl)
    pltpu.sync_copy(m_hbm.at[pl.ds(off0, TPT)], msl)
    pltpu.sync_copy(tm1_hbm.at[pl.ds(off0, TPT)], tsl)

    def _u(i, c):
        ds = pl.ds(i * L, L)
        syn = p0[ds] + p1[ds]
        vn = vsl[ds] * DECAY + syn * DT
        m = msl[ds]
        vn = vn * (1.0 - m) + tsl[ds] * m
        vsl[ds] = vn
        return c

    lax.fori_loop(0, TPT // L, _u, 0)
    pltpu.sync_copy(vsl, v_out.at[pl.ds(off0, TPT)])


_update_call = functools.partial(
    pl.kernel,
    out_type=jax.ShapeDtypeStruct((NPAD,), jnp.float32),
    mesh=plsc.VectorSubcoreMesh(
        core_axis_name="c", subcore_axis_name="s", num_cores=NC,
        num_subcores=NS,
    ),
    compiler_params=pltpu.CompilerParams(needs_layout_passes=False),
    scratch_types=(
        pltpu.VMEM((TPT,), jnp.float32),
        pltpu.VMEM((TPT,), jnp.float32),
        pltpu.VMEM((TPT,), jnp.float32),
        pltpu.VMEM((TPT,), jnp.float32),
        pltpu.VMEM((TPT,), jnp.float32),
    ),
)(_update)


def kernel(tm1_input, weights, source_indices, target_indices, type_ids, steps):
    e = weights.shape[0]
    step_call = _make_step(e)

    tm1_idx = jnp.nonzero(type_ids == 0, size=tm1_input.shape[1])[0]
    m = jnp.zeros((NPAD,), jnp.float32).at[tm1_idx].set(1.0)
    tm1f = jnp.zeros((NPAD,), jnp.float32).at[tm1_idx].set(tm1_input[0])
    v_init = jnp.zeros((NPAD,), jnp.float32)
    p_init = jnp.zeros((NC * NPAD,), jnp.float32)

    def body(_, carry):
        v, p = carry
        v_new, p_new, _ = step_call(
            source_indices, target_indices, weights, p, m, tm1f, v)
        return v_new, p_new

    v_last, p_last = lax.fori_loop(0, steps, body, (v_init, p_init))
    v_fin = _update_call(p_last, m, tm1f, v_last)
    return v_fin[:N_NEURONS].reshape(1, N_NEURONS)


# speedup vs baseline: 7.7505x; 1.1395x over previous
import functools

import jax
import jax.numpy as jnp
from jax import lax
from jax.experimental import pallas as pl
from jax.experimental.pallas import tpu as pltpu
from jax.experimental.pallas import tpu_sc as plsc

N_NEURONS = 100000
DT = 0.1
DECAY = 1.0 - DT

NC = 2
NS = 16
NW = NC * NS
L = 16

TPT = 3136
NPAD = NW * TPT
SPT = NPAD // NS
CHUNK = 1792
UNROLL = 16


def _make_step(e):
    assert e % NW == 0 and (e // NW) % L == 0
    pt = e // NW
    nch = -(-pt // CHUNK)
    assert nch >= 3 and pt - (nch - 1) * CHUNK >= L

    def body(s_hbm, t_hbm, w_hbm, p_hbm, m_hbm, tm1_hbm, v_hbm,
             v_out, p_out, rb_hbm,
             r_full, acc, sb0, tb0, wb0, cb0, sb1, tb1, wb1, cb1,
             sb2, tb2, wb2, cb2,
             semr, semi0, semi1, semi2, semo0, semo1, semo2):
        cid = lax.axis_index("c")
        sid = lax.axis_index("s")
        wid = cid * NS + sid
        base = pl.multiple_of(wid * pt, 8)
        uoff = pl.multiple_of(sid * SPT, 8)

        P0, P1, VV, MM, TT, RR, ZZ = (
            r_full.at[pl.ds(k * SPT, SPT)] for k in range(7))

        pltpu.make_async_copy(p_hbm.at[pl.ds(uoff, SPT)], P0, semr).start()
        pltpu.make_async_copy(
            p_hbm.at[pl.ds(pl.multiple_of(NPAD + sid * SPT, 8), SPT)],
            P1, semr).start()
        pltpu.make_async_copy(v_hbm.at[pl.ds(uoff, SPT)], VV, semr).start()
        pltpu.make_async_copy(m_hbm.at[pl.ds(uoff, SPT)], MM, semr).start()
        pltpu.make_async_copy(tm1_hbm.at[pl.ds(uoff, SPT)], TT, semr).start()

        zeros = jnp.zeros((L,), jnp.float32)

        def _z(i, c):
            ZZ[pl.ds(i * L, L)] = zeros
            return c

        lax.fori_loop(0, SPT // L, _z, 0)
        pltpu.sync_copy(ZZ, acc.at[pl.ds(uoff, SPT)])

        pltpu.make_async_copy(p_hbm.at[pl.ds(uoff, SPT)], P0, semr).wait()
        pltpu.make_async_copy(
            p_hbm.at[pl.ds(pl.multiple_of(NPAD + sid * SPT, 8), SPT)],
            P1, semr).wait()
        pltpu.make_async_copy(v_hbm.at[pl.ds(uoff, SPT)], VV, semr).wait()
        pltpu.make_async_copy(m_hbm.at[pl.ds(uoff, SPT)], MM, semr).wait()
        pltpu.make_async_copy(tm1_hbm.at[pl.ds(uoff, SPT)], TT, semr).wait()

        def _u(i, c):
            ds = pl.ds(i * L, L)
            syn = P0[ds] + P1[ds]
            vn = VV[ds] * DECAY + syn * DT
            m = MM[ds]
            vn = vn * (1.0 - m) + TT[ds] * m
            VV[ds] = vn
            RR[ds] = jnp.maximum(vn, 0.0)
            return c

        lax.fori_loop(0, SPT // L, _u, 0)

        @pl.when(cid == 0)
        def _():
            pltpu.sync_copy(VV, v_out.at[pl.ds(uoff, SPT)])

        roff = pl.multiple_of(cid * NPAD + sid * SPT, 8)
        pltpu.sync_copy(RR, rb_hbm.at[pl.ds(roff, SPT)])
        plsc.subcore_barrier()

        rcp = pltpu.make_async_copy(
            rb_hbm.at[pl.ds(pl.multiple_of(cid * NPAD, 8), NPAD)],
            r_full, semr)
        rcp.start()

        bufs = (
            (sb0, tb0, wb0, cb0, semi0, semo0),
            (sb1, tb1, wb1, cb1, semi1, semo1),
            (sb2, tb2, wb2, cb2, semi2, semo2),
        )

        def start_in(j, bb):
            sb, tb, wb, _, semi, _ = bb
            off = pl.multiple_of(base + j * CHUNK, 8)
            n = min(CHUNK, pt - j * CHUNK)
            pltpu.make_async_copy(s_hbm.at[pl.ds(off, n)],
                                  sb.at[pl.ds(0, n)], semi).start()
            pltpu.make_async_copy(t_hbm.at[pl.ds(off, n)],
                                  tb.at[pl.ds(0, n)], semi).start()
            pltpu.make_async_copy(w_hbm.at[pl.ds(off, n)],
                                  wb.at[pl.ds(0, n)], semi).start()

        def wait_in(j, bb):
            sb, tb, wb, _, semi, _ = bb
            off = pl.multiple_of(base + j * CHUNK, 8)
            n = min(CHUNK, pt - j * CHUNK)
            pltpu.make_async_copy(s_hbm.at[pl.ds(off, n)],
                                  sb.at[pl.ds(0, n)], semi).wait()
            pltpu.make_async_copy(t_hbm.at[pl.ds(off, n)],
                                  tb.at[pl.ds(0, n)], semi).wait()
            pltpu.make_async_copy(w_hbm.at[pl.ds(off, n)],
                                  wb.at[pl.ds(0, n)], semi).wait()

        def compute(j, bb):
            sb, _, wb, cb, _, _ = bb
            n = min(CHUNK, pt - j * CHUNK)
            nv = n // L

            def _g(k, c):
                o = k * (UNROLL * L)
                for u in range(UNROLL):
                    oo = o + u * L
                    sv = sb[pl.ds(oo, L)]
                    wv = wb[pl.ds(oo, L)]
                    vals = plsc.load_gather(r_full, [sv])
                    cb[pl.ds(oo, L)] = vals * wv
                return c

            lax.fori_loop(0, nv // UNROLL, _g, 0)
            for u in range(nv - (nv // UNROLL) * UNROLL):
                oo = (nv // UNROLL) * UNROLL * L + u * L
                sv = sb[pl.ds(oo, L)]
                wv = wb[pl.ds(oo, L)]
                vals = plsc.load_gather(r_full, [sv])
                cb[pl.ds(oo, L)] = vals * wv
            if n < CHUNK:
                def _zt(k, c):
                    cb[pl.ds(n + k * L, L)] = zeros
                    return c

                lax.fori_loop(0, (CHUNK - n) // L, _zt, 0)

        def start_scat(bb):
            _, tb, _, cb, _, semo = bb
            pltpu.async_copy(cb, acc.at[tb], semo, add=True)

        def wait_scat(bb):
            _, tb, _, cb, _, semo = bb
            pltpu.make_async_copy(cb, acc.at[tb], semo).wait()

        start_in(0, bufs[0])
        rcp.wait()
        for j in range(nch):
            cur = bufs[j % 3]
            wait_in(j, cur)
            if j + 1 < nch:
                if j >= 2:
                    wait_scat(bufs[(j - 2) % 3])
                start_in(j + 1, bufs[(j + 1) % 3])
            compute(j, cur)
            start_scat(cur)
        wait_scat(bufs[(nch - 3) % 3])
        wait_scat(bufs[(nch - 2) % 3])
        wait_scat(bufs[(nch - 1) % 3])

        plsc.subcore_barrier()
        poff = pl.multiple_of(cid * NPAD + sid * SPT, 8)
        pltpu.sync_copy(acc.at[pl.ds(uoff, SPT)], p_out.at[pl.ds(poff, SPT)])

    return pl.kernel(
        body,
        out_type=(
            jax.ShapeDtypeStruct((NPAD,), jnp.float32),
            jax.ShapeDtypeStruct((NC * NPAD,), jnp.float32),
            jax.ShapeDtypeStruct((NC * NPAD,), jnp.float32),
        ),
        mesh=plsc.VectorSubcoreMesh(
            core_axis_name="c", subcore_axis_name="s", num_cores=NC,
            num_subcores=NS,
        ),
        compiler_params=pltpu.CompilerParams(needs_layout_passes=False),
        scratch_types=(
            pltpu.VMEM((NPAD,), jnp.float32),
            pltpu.VMEM_SHARED((NPAD,), jnp.float32),
            pltpu.VMEM((CHUNK,), jnp.int32),
            pltpu.VMEM((CHUNK,), jnp.int32),
            pltpu.VMEM((CHUNK,), jnp.float32),
            pltpu.VMEM((CHUNK,), jnp.float32),
            pltpu.VMEM((CHUNK,), jnp.int32),
            pltpu.VMEM((CHUNK,), jnp.int32),
            pltpu.VMEM((CHUNK,), jnp.float32),
            pltpu.VMEM((CHUNK,), jnp.float32),
            pltpu.VMEM((CHUNK,), jnp.int32),
            pltpu.VMEM((CHUNK,), jnp.int32),
            pltpu.VMEM((CHUNK,), jnp.float32),
            pltpu.VMEM((CHUNK,), jnp.float32),
            pltpu.SemaphoreType.DMA,
            pltpu.SemaphoreType.DMA,
            pltpu.SemaphoreType.DMA,
            pltpu.SemaphoreType.DMA,
            pltpu.SemaphoreType.DMA,
            pltpu.SemaphoreType.DMA,
            pltpu.SemaphoreType.DMA,
        ),
    )


def _update(p_hbm, m_hbm, tm1_hbm, v_hbm, v_out,
            p0, p1, vsl, msl, tsl):
    wid = lax.axis_index("c") * NS + lax.axis_index("s")
    off0 = pl.multiple_of(wid * TPT, 8)
    pltpu.sync_copy(p_hbm.at[pl.ds(off0, TPT)], p0)
    pltpu.sync_copy(p_hbm.at[pl.ds(pl.multiple_of(NPAD + wid * TPT, 8), TPT)],
                    p1)
    pltpu.sync_copy(v_hbm.at[pl.ds(off0, TPT)], vsl)
    pltpu.sync_copy(m_hbm.at[pl.ds(off0, TPT)], msl)
    pltpu.sync_copy(tm1_hbm.at[pl.ds(off0, TPT)], tsl)

    def _u(i, c):
        ds = pl.ds(i * L, L)
        syn = p0[ds] + p1[ds]
        vn = vsl[ds] * DECAY + syn * DT
        m = msl[ds]
        vn = vn * (1.0 - m) + tsl[ds] * m
        vsl[ds] = vn
        return c

    lax.fori_loop(0, TPT // L, _u, 0)
    pltpu.sync_copy(vsl, v_out.at[pl.ds(off0, TPT)])


_update_call = functools.partial(
    pl.kernel,
    out_type=jax.ShapeDtypeStruct((NPAD,), jnp.float32),
    mesh=plsc.VectorSubcoreMesh(
        core_axis_name="c", subcore_axis_name="s", num_cores=NC,
        num_subcores=NS,
    ),
    compiler_params=pltpu.CompilerParams(needs_layout_passes=False),
    scratch_types=(
        pltpu.VMEM((TPT,), jnp.float32),
        pltpu.VMEM((TPT,), jnp.float32),
        pltpu.VMEM((TPT,), jnp.float32),
        pltpu.VMEM((TPT,), jnp.float32),
        pltpu.VMEM((TPT,), jnp.float32),
    ),
)(_update)


def kernel(tm1_input, weights, source_indices, target_indices, type_ids, steps):
    e = weights.shape[0]
    step_call = _make_step(e)

    tm1_idx = jnp.nonzero(type_ids == 0, size=tm1_input.shape[1])[0]
    m = jnp.zeros((NPAD,), jnp.float32).at[tm1_idx].set(1.0)
    tm1f = jnp.zeros((NPAD,), jnp.float32).at[tm1_idx].set(tm1_input[0])
    v_init = jnp.zeros((NPAD,), jnp.float32)
    p_init = jnp.zeros((NC * NPAD,), jnp.float32)

    def body(_, carry):
        v, p = carry
        v_new, p_new, _ = step_call(
            source_indices, target_indices, weights, p, m, tm1f, v)
        return v_new, p_new

    v_last, p_last = lax.fori_loop(0, steps, body, (v_init, p_init))
    v_fin = _update_call(p_last, m, tm1f, v_last)
    return v_fin[:N_NEURONS].reshape(1, N_NEURONS)
